# Initial kernel scaffold; baseline (speedup 1.0000x reference)
import jax
import jax.numpy as jnp
from jax.experimental import pallas as pl


def kernel(features, xyz, W_fc1, b_fc1, W_c2, b_c2, W_c3, b_c3, W_d1, b_d1, W_d2, b_d2, W_g1, b_g1, W_g2, b_g2, W_q, W_k, W_v):
    raise NotImplementedError("write your pallas kernel here")


# speedup vs baseline: 10.4870x; 10.4870x over previous
import functools

import jax
import jax.numpy as jnp
from jax import lax
from jax.experimental import pallas as pl
from jax.experimental.pallas import tpu as pltpu
from jax.experimental.pallas import tpu_sc as plsc

B, N, D_POINTS, D_MODEL, K = 4, 1024, 32, 128, 36
BN = B * N
PAIRS = BN * K
R = 256
M = 128
PB = M * K
NC, NS = 2, 16
NW = NC * NS
PER_W = PAIRS // NW
CHUNK = 128
N_CHUNKS = PER_W // CHUNK

_f32 = jnp.float32
_dn = lambda c_lhs, c_rhs: (((c_lhs,), (c_rhs,)), ((), ()))


def _mm(a, b, cl=1, cr=1):
    return lax.dot_general(a, b, _dn(cl, cr), preferred_element_type=_f32)


def _proj_body(feat_ref, xyz_ref, wfc1_ref, bfc1_ref, wq_ref, wk_ref, wv_ref,
               wd1_ref, bd1_ref, x_ref, q_ref, kvq_ref, p_ref):
    feat = feat_ref[0]
    x = _mm(feat, wfc1_ref[...]) + bfc1_ref[...]
    q = _mm(x, wq_ref[...])
    k = _mm(x, wk_ref[...])
    v = _mm(x, wv_ref[...])
    x_ref[0] = x
    q_ref[0] = q
    xyz = xyz_ref[0]
    xyzmm = _mm(xyz, wd1_ref[...])
    kvq_ref[0] = jnp.concatenate([k, v, xyzmm], axis=1)
    p_ref[0] = xyzmm + bd1_ref[...]


def _run_proj(features, xyzp, wfc1, bfc1, wq, wk, wv, wd1p, bd1):
    spec_w = lambda shp: pl.BlockSpec(shp, lambda b: (0,) * len(shp))
    return pl.pallas_call(
        _proj_body,
        grid=(B,),
        in_specs=[
            pl.BlockSpec((1, N, D_POINTS), lambda b: (b, 0, 0)),
            pl.BlockSpec((1, N, 4), lambda b: (b, 0, 0)),
            spec_w((D_MODEL, D_POINTS)),
            spec_w((1, D_MODEL)),
            spec_w((D_MODEL, D_MODEL)),
            spec_w((D_MODEL, D_MODEL)),
            spec_w((D_MODEL, D_MODEL)),
            spec_w((D_MODEL, 4)),
            spec_w((1, D_MODEL)),
        ],
        out_specs=[
            pl.BlockSpec((1, N, D_MODEL), lambda b: (b, 0, 0)),
            pl.BlockSpec((1, N, D_MODEL), lambda b: (b, 0, 0)),
            pl.BlockSpec((1, N, 3 * D_MODEL), lambda b: (b, 0, 0)),
            pl.BlockSpec((1, N, D_MODEL), lambda b: (b, 0, 0)),
        ],
        out_shape=[
            jax.ShapeDtypeStruct((B, N, D_MODEL), _f32),
            jax.ShapeDtypeStruct((B, N, D_MODEL), _f32),
            jax.ShapeDtypeStruct((B, N, 3 * D_MODEL), _f32),
            jax.ShapeDtypeStruct((B, N, D_MODEL), _f32),
        ],
    )(features, xyzp, wfc1, bfc1, wq, wk, wv, wd1p, bd1)


def _topk_body(x_rows_ref, x_all_ref, out_ref):
    b = pl.program_id(0)
    xr = x_rows_ref[0]
    xa = x_all_ref[0]
    t = lax.dot_general(xr.astype(jnp.bfloat16), xa.astype(jnp.bfloat16),
                        _dn(1, 1), preferred_element_type=_f32)
    sqr = jnp.sum(xr * xr, axis=1, keepdims=True)
    sqa = jnp.reshape(jnp.sum(xa * xa, axis=1, keepdims=True), (1, N))
    d = (-2.0 * t + sqr) + sqa
    iota = lax.broadcasted_iota(jnp.int32, (R, N), 1)
    base = b * N
    for t in range(K):
        m = jnp.min(d, axis=1, keepdims=True)
        idx = jnp.min(jnp.where(d == m, iota, N), axis=1, keepdims=True)
        out_ref[0, :, pl.ds(t, 1)] = idx + base
        d = jnp.where(iota == idx, 3.4e38, d)


def _run_topk(xyzp):
    return pl.pallas_call(
        _topk_body,
        grid=(B, N // R),
        in_specs=[
            pl.BlockSpec((1, R, 4), lambda b, rb: (b, rb, 0)),
            pl.BlockSpec((1, N, 4), lambda b, rb: (b, 0, 0)),
        ],
        out_specs=pl.BlockSpec((1, R, K), lambda b, rb: (b, rb, 0)),
        out_shape=jax.ShapeDtypeStruct((B, N, K), jnp.int32),
    )(xyzp, xyzp)


def _sc_gather_body(kvq_hbm, idx_hbm, kvg_out, idx_v, rows_v, sem1):
    c = lax.axis_index("c")
    s = lax.axis_index("s")
    wid = s * NC + c
    base = wid * PER_W

    def step(j, carry):
        off = pl.multiple_of(base + j * CHUNK, CHUNK)
        pltpu.sync_copy(idx_hbm.at[pl.ds(off, CHUNK)], idx_v)
        pltpu.async_copy(kvq_hbm.at[idx_v], rows_v, sem1).wait()
        pltpu.sync_copy(rows_v, kvg_out.at[pl.ds(off, CHUNK)])
        return carry

    lax.fori_loop(0, N_CHUNKS, step, 0)


def _run_sc_gather(kvq_tab, idx_flat):
    mesh = plsc.VectorSubcoreMesh(core_axis_name="c", subcore_axis_name="s")
    return pl.kernel(
        _sc_gather_body,
        out_type=jax.ShapeDtypeStruct((PAIRS, 3 * D_MODEL), _f32),
        mesh=mesh,
        scratch_types=[
            pltpu.VMEM((CHUNK,), jnp.int32),
            pltpu.VMEM((CHUNK, 3 * D_MODEL), _f32),
            pltpu.SemaphoreType.DMA,
        ],
    )(kvq_tab, idx_flat)


def _pair_body(q_ref, p_ref, kvg_ref, wd2_ref,
               bd2_ref, wg1_ref, bg1_ref, wg2_ref, bg2_ref, attn_ref, res_ref):
    kvg = kvg_ref[...]
    kg = kvg[:, :D_MODEL]
    vg = kvg[:, D_MODEL:2 * D_MODEL]
    qg = kvg[:, 2 * D_MODEL:]
    p_rep = jnp.reshape(
        jnp.broadcast_to(p_ref[...][:, None, :], (M, K, D_MODEL)),
        (PB, D_MODEL))
    h1 = jnp.maximum(p_rep - qg, 0.0)
    pos = _mm(h1, wd2_ref[...]) + bd2_ref[...]
    q_rep = jnp.reshape(
        jnp.broadcast_to(q_ref[...][:, None, :], (M, K, D_MODEL)),
        (PB, D_MODEL))
    t = q_rep - kg + pos
    h2 = jnp.maximum(_mm(t, wg1_ref[...]) + bg1_ref[...], 0.0)
    attn = _mm(h2, wg2_ref[...]) + bg2_ref[...]
    den = jnp.sum(jnp.abs(attn) + 1e-5, axis=1, keepdims=True)
    attn = attn / den
    attn_ref[...] = attn
    w = attn * (vg + pos)
    res_ref[...] = jnp.sum(jnp.reshape(w, (M, K, D_MODEL)), axis=1)


def _run_pair(q2, p2, kvg, wd2, bd2, wg1, bg1, wg2, bg2):
    spec_w = lambda shp: pl.BlockSpec(shp, lambda i: (0,) * len(shp))
    return pl.pallas_call(
        _pair_body,
        grid=(BN // M,),
        in_specs=[
            pl.BlockSpec((M, D_MODEL), lambda i: (i, 0)),
            pl.BlockSpec((M, D_MODEL), lambda i: (i, 0)),
            pl.BlockSpec((PB, 3 * D_MODEL), lambda i: (i, 0)),
            spec_w((D_MODEL, D_MODEL)),
            spec_w((1, D_MODEL)),
            spec_w((D_MODEL, D_MODEL)),
            spec_w((1, D_MODEL)),
            spec_w((D_MODEL, D_MODEL)),
            spec_w((1, D_MODEL)),
        ],
        out_specs=[
            pl.BlockSpec((PB, D_MODEL), lambda i: (i, 0)),
            pl.BlockSpec((M, D_MODEL), lambda i: (i, 0)),
        ],
        out_shape=[
            jax.ShapeDtypeStruct((PAIRS, D_MODEL), _f32),
            jax.ShapeDtypeStruct((BN, D_MODEL), _f32),
        ],
    )(q2, p2, kvg, wd2, bd2, wg1, bg1, wg2, bg2)


def _mt_body(resp_ref, x_ref, wc2_ref, bc2_ref, wc3_ref, bc3_ref, out_ref):
    feat0 = resp_ref[0]
    x = x_ref[0]
    ones = jnp.ones((1, N), _f32)

    def mtl(feat, wc, bc):
        m = _mm(ones, feat, 1, 0) * (1.0 / N)
        wv = _mm(m, wc)
        g = jnp.reshape(wv, (D_MODEL, D_MODEL)) + bc
        s = jnp.sum(jnp.abs(g) + 1e-5, axis=0, keepdims=True)
        return _mm(feat * (1.0 / s), g)

    r1 = mtl(feat0, wc2_ref[...], bc2_ref[...]) + x
    r2 = mtl(r1, wc3_ref[...], bc3_ref[...]) + r1
    out_ref[0] = r2


def _run_mt(res_pre, x, wc2, bc2r, wc3, bc3r):
    spec_w = lambda shp: pl.BlockSpec(shp, lambda b: (0,) * len(shp))
    return pl.pallas_call(
        _mt_body,
        grid=(B,),
        in_specs=[
            pl.BlockSpec((1, N, D_MODEL), lambda b: (b, 0, 0)),
            pl.BlockSpec((1, N, D_MODEL), lambda b: (b, 0, 0)),
            spec_w((D_MODEL * D_MODEL, D_MODEL)),
            spec_w((D_MODEL, D_MODEL)),
            spec_w((D_MODEL * D_MODEL, D_MODEL)),
            spec_w((D_MODEL, D_MODEL)),
        ],
        out_specs=pl.BlockSpec((1, N, D_MODEL), lambda b: (b, 0, 0)),
        out_shape=jax.ShapeDtypeStruct((B, N, D_MODEL), _f32),
    )(res_pre, x, wc2, bc2r, wc3, bc3r)


def kernel(features, xyz, W_fc1, b_fc1, W_c2, b_c2, W_c3, b_c3, W_d1, b_d1,
           W_d2, b_d2, W_g1, b_g1, W_g2, b_g2, W_q, W_k, W_v):
    xyzp = jnp.pad(xyz, ((0, 0), (0, 0), (0, 1)))
    wd1p = jnp.pad(W_d1, ((0, 0), (0, 1)))
    x, q, kvq, p = _run_proj(
        features, xyzp, W_fc1, b_fc1.reshape(1, -1), W_q, W_k, W_v,
        wd1p, b_d1.reshape(1, -1))

    knn = _run_topk(xyzp)
    idx_flat = knn.reshape(PAIRS)

    kvqg = _run_sc_gather(kvq.reshape(BN, 3 * D_MODEL), idx_flat)

    attn, res_pre = _run_pair(
        q.reshape(BN, D_MODEL), p.reshape(BN, D_MODEL), kvqg,
        W_d2, b_d2.reshape(1, -1),
        W_g1, b_g1.reshape(1, -1), W_g2, b_g2.reshape(1, -1))

    res = _run_mt(res_pre.reshape(B, N, D_MODEL), x, W_c2,
                  b_c2.reshape(D_MODEL, D_MODEL), W_c3,
                  b_c3.reshape(D_MODEL, D_MODEL))

    return (res, attn.reshape(B, N, K, D_MODEL))


# speedup vs baseline: 11.9502x; 1.1395x over previous
import functools

import jax
import jax.numpy as jnp
from jax import lax
from jax.experimental import pallas as pl
from jax.experimental.pallas import tpu as pltpu
from jax.experimental.pallas import tpu_sc as plsc

B, N, D_POINTS, D_MODEL, K = 4, 1024, 32, 128, 36
BN = B * N
PAIRS = BN * K
R = 256
M = 128
PB = M * K
NC, NS = 2, 16
NW = NC * NS
PER_W = PAIRS // NW
CHUNK = 128
N_CHUNKS = PER_W // CHUNK

_f32 = jnp.float32
_dn = lambda c_lhs, c_rhs: (((c_lhs,), (c_rhs,)), ((), ()))


def _mm(a, b, cl=1, cr=1):
    return lax.dot_general(a, b, _dn(cl, cr), preferred_element_type=_f32)


def _proj_body(feat_ref, xyz_ref, wfc1_ref, bfc1_ref, wq_ref, wk_ref, wv_ref,
               wd1_ref, bd1_ref, x_ref, q_ref, kvq_ref, p_ref):
    feat = feat_ref[0]
    x = _mm(feat, wfc1_ref[...]) + bfc1_ref[...]
    q = _mm(x, wq_ref[...])
    k = _mm(x, wk_ref[...])
    v = _mm(x, wv_ref[...])
    x_ref[0] = x
    q_ref[0] = q
    xyz = xyz_ref[0]
    xyzmm = _mm(xyz, wd1_ref[...])
    kvq_ref[0] = jnp.concatenate([k, v, xyzmm], axis=1)
    p_ref[0] = xyzmm + bd1_ref[...]


def _run_proj(features, xyzp, wfc1, bfc1, wq, wk, wv, wd1p, bd1):
    spec_w = lambda shp: pl.BlockSpec(shp, lambda b: (0,) * len(shp))
    return pl.pallas_call(
        _proj_body,
        grid=(B,),
        in_specs=[
            pl.BlockSpec((1, N, D_POINTS), lambda b: (b, 0, 0)),
            pl.BlockSpec((1, N, 4), lambda b: (b, 0, 0)),
            spec_w((D_MODEL, D_POINTS)),
            spec_w((1, D_MODEL)),
            spec_w((D_MODEL, D_MODEL)),
            spec_w((D_MODEL, D_MODEL)),
            spec_w((D_MODEL, D_MODEL)),
            spec_w((D_MODEL, 4)),
            spec_w((1, D_MODEL)),
        ],
        out_specs=[
            pl.BlockSpec((1, N, D_MODEL), lambda b: (b, 0, 0)),
            pl.BlockSpec((1, N, D_MODEL), lambda b: (b, 0, 0)),
            pl.BlockSpec((1, N, 3 * D_MODEL), lambda b: (b, 0, 0)),
            pl.BlockSpec((1, N, D_MODEL), lambda b: (b, 0, 0)),
        ],
        out_shape=[
            jax.ShapeDtypeStruct((B, N, D_MODEL), _f32),
            jax.ShapeDtypeStruct((B, N, D_MODEL), _f32),
            jax.ShapeDtypeStruct((B, N, 3 * D_MODEL), _f32),
            jax.ShapeDtypeStruct((B, N, D_MODEL), _f32),
        ],
    )(features, xyzp, wfc1, bfc1, wq, wk, wv, wd1p, bd1)


def _topk_body(x_rows_ref, x_all_ref, out_ref):
    b = pl.program_id(0)
    xr = x_rows_ref[0]
    xa = x_all_ref[0]
    t = lax.dot_general(xr.astype(jnp.bfloat16), xa.astype(jnp.bfloat16),
                        _dn(1, 1), preferred_element_type=_f32)
    sqr = jnp.sum(xr * xr, axis=1, keepdims=True)
    sqa = jnp.reshape(jnp.sum(xa * xa, axis=1, keepdims=True), (1, N))
    d = (-2.0 * t + sqr) + sqa
    iota = lax.broadcasted_iota(jnp.int32, (R, N), 1)
    base = b * N
    for t in range(K):
        m = jnp.min(d, axis=1, keepdims=True)
        sel = jnp.where(d == m, iota, N)
        idx = jnp.min(sel, axis=1, keepdims=True)
        out_ref[0, :, pl.ds(t, 1)] = idx + base
        d = jnp.where(sel == idx, 3.4e38, d)


def _run_topk(xyzp):
    return pl.pallas_call(
        _topk_body,
        grid=(B, N // R),
        in_specs=[
            pl.BlockSpec((1, R, 4), lambda b, rb: (b, rb, 0)),
            pl.BlockSpec((1, N, 4), lambda b, rb: (b, 0, 0)),
        ],
        out_specs=pl.BlockSpec((1, R, K), lambda b, rb: (b, rb, 0)),
        out_shape=jax.ShapeDtypeStruct((B, N, K), jnp.int32),
    )(xyzp, xyzp)


def _sc_gather_body(kvq_hbm, idx_hbm, kvg_out, idx_v, rows_a, rows_b,
                    sem_a, sem_b, semw_a, semw_b):
    c = lax.axis_index("c")
    s = lax.axis_index("s")
    wid = s * NC + c
    base = wid * PER_W

    pltpu.sync_copy(idx_hbm.at[pl.ds(base, PER_W)], idx_v)
    bufs = [(rows_a, sem_a, semw_a), (rows_b, sem_b, semw_b)]

    def start(j):
        buf, sem, _ = bufs[j % 2]
        return pltpu.async_copy(
            kvq_hbm.at[idx_v.at[pl.ds(j * CHUNK, CHUNK)]], buf, sem)

    cps = {0: start(0), 1: start(1)}
    wrs = {}
    for j in range(N_CHUNKS):
        buf, _, semw = bufs[j % 2]
        cps[j].wait()
        wrs[j] = pltpu.async_copy(
            buf, kvg_out.at[pl.ds(base + j * CHUNK, CHUNK)], semw)
        if j + 2 < N_CHUNKS:
            wrs[j].wait()
            cps[j + 2] = start(j + 2)
    wrs[N_CHUNKS - 2].wait()
    wrs[N_CHUNKS - 1].wait()


def _run_sc_gather(kvq_tab, idx_flat):
    mesh = plsc.VectorSubcoreMesh(core_axis_name="c", subcore_axis_name="s")
    return pl.kernel(
        _sc_gather_body,
        out_type=jax.ShapeDtypeStruct((PAIRS, 3 * D_MODEL), _f32),
        mesh=mesh,
        scratch_types=[
            pltpu.VMEM((PER_W,), jnp.int32),
            pltpu.VMEM((CHUNK, 3 * D_MODEL), _f32),
            pltpu.VMEM((CHUNK, 3 * D_MODEL), _f32),
            pltpu.SemaphoreType.DMA,
            pltpu.SemaphoreType.DMA,
            pltpu.SemaphoreType.DMA,
            pltpu.SemaphoreType.DMA,
        ],
    )(kvq_tab, idx_flat)


def _pair_body(q_ref, p_ref, kvg_ref, wd2_ref,
               bd2_ref, wg1_ref, bg1_ref, wg2_ref, bg2_ref, attn_ref, res_ref):
    kvg = kvg_ref[...]
    kg = kvg[:, :D_MODEL]
    vg = kvg[:, D_MODEL:2 * D_MODEL]
    qg = kvg[:, 2 * D_MODEL:]
    p_rep = jnp.reshape(
        jnp.broadcast_to(p_ref[...][:, None, :], (M, K, D_MODEL)),
        (PB, D_MODEL))
    h1 = jnp.maximum(p_rep - qg, 0.0)
    pos = _mm(h1, wd2_ref[...]) + bd2_ref[...]
    q_rep = jnp.reshape(
        jnp.broadcast_to(q_ref[...][:, None, :], (M, K, D_MODEL)),
        (PB, D_MODEL))
    t = q_rep - kg + pos
    h2 = jnp.maximum(_mm(t, wg1_ref[...]) + bg1_ref[...], 0.0)
    attn = _mm(h2, wg2_ref[...]) + bg2_ref[...]
    den = jnp.sum(jnp.abs(attn) + 1e-5, axis=1, keepdims=True)
    attn = attn / den
    attn_ref[0] = jnp.reshape(attn, (M, K, D_MODEL))
    w = attn * (vg + pos)
    res_ref[...] = jnp.sum(jnp.reshape(w, (M, K, D_MODEL)), axis=1)


def _run_pair(q2, p2, kvg, wd2, bd2, wg1, bg1, wg2, bg2):
    spec_w = lambda shp: pl.BlockSpec(shp, lambda i: (0,) * len(shp))
    return pl.pallas_call(
        _pair_body,
        grid=(BN // M,),
        in_specs=[
            pl.BlockSpec((M, D_MODEL), lambda i: (i, 0)),
            pl.BlockSpec((M, D_MODEL), lambda i: (i, 0)),
            pl.BlockSpec((PB, 3 * D_MODEL), lambda i: (i, 0)),
            spec_w((D_MODEL, D_MODEL)),
            spec_w((1, D_MODEL)),
            spec_w((D_MODEL, D_MODEL)),
            spec_w((1, D_MODEL)),
            spec_w((D_MODEL, D_MODEL)),
            spec_w((1, D_MODEL)),
        ],
        out_specs=[
            pl.BlockSpec((1, M, K, D_MODEL),
                         lambda i: (i // (N // M), i % (N // M), 0, 0)),
            pl.BlockSpec((M, D_MODEL), lambda i: (i, 0)),
        ],
        out_shape=[
            jax.ShapeDtypeStruct((B, N, K, D_MODEL), _f32),
            jax.ShapeDtypeStruct((BN, D_MODEL), _f32),
        ],
    )(q2, p2, kvg, wd2, bd2, wg1, bg1, wg2, bg2)


def _mt_body(resp_ref, x_ref, wc2_ref, bc2_ref, wc3_ref, bc3_ref, out_ref):
    feat0 = resp_ref[0]
    x = x_ref[0]
    ones = jnp.ones((1, N), _f32)

    def mtl(feat, wc, bc):
        m = _mm(ones, feat, 1, 0) * (1.0 / N)
        wv = _mm(m, wc)
        g = jnp.reshape(wv, (D_MODEL, D_MODEL)) + bc
        s = jnp.sum(jnp.abs(g) + 1e-5, axis=0, keepdims=True)
        return _mm(feat * (1.0 / s), g)

    r1 = mtl(feat0, wc2_ref[...], bc2_ref[...]) + x
    r2 = mtl(r1, wc3_ref[...], bc3_ref[...]) + r1
    out_ref[0] = r2


def _run_mt(res_pre, x, wc2, bc2r, wc3, bc3r):
    spec_w = lambda shp: pl.BlockSpec(shp, lambda b: (0,) * len(shp))
    return pl.pallas_call(
        _mt_body,
        grid=(B,),
        in_specs=[
            pl.BlockSpec((1, N, D_MODEL), lambda b: (b, 0, 0)),
            pl.BlockSpec((1, N, D_MODEL), lambda b: (b, 0, 0)),
            spec_w((D_MODEL * D_MODEL, D_MODEL)),
            spec_w((D_MODEL, D_MODEL)),
            spec_w((D_MODEL * D_MODEL, D_MODEL)),
            spec_w((D_MODEL, D_MODEL)),
        ],
        out_specs=pl.BlockSpec((1, N, D_MODEL), lambda b: (b, 0, 0)),
        out_shape=jax.ShapeDtypeStruct((B, N, D_MODEL), _f32),
    )(res_pre, x, wc2, bc2r, wc3, bc3r)


def kernel(features, xyz, W_fc1, b_fc1, W_c2, b_c2, W_c3, b_c3, W_d1, b_d1,
           W_d2, b_d2, W_g1, b_g1, W_g2, b_g2, W_q, W_k, W_v):
    xyzp = jnp.pad(xyz, ((0, 0), (0, 0), (0, 1)))
    wd1p = jnp.pad(W_d1, ((0, 0), (0, 1)))
    x, q, kvq, p = _run_proj(
        features, xyzp, W_fc1, b_fc1.reshape(1, -1), W_q, W_k, W_v,
        wd1p, b_d1.reshape(1, -1))

    knn = _run_topk(xyzp)
    idx_flat = knn.reshape(PAIRS)

    kvqg = _run_sc_gather(kvq.reshape(BN, 3 * D_MODEL), idx_flat)

    attn, res_pre = _run_pair(
        q.reshape(BN, D_MODEL), p.reshape(BN, D_MODEL), kvqg,
        W_d2, b_d2.reshape(1, -1),
        W_g1, b_g1.reshape(1, -1), W_g2, b_g2.reshape(1, -1))

    res = _run_mt(res_pre.reshape(B, N, D_MODEL), x, W_c2,
                  b_c2.reshape(D_MODEL, D_MODEL), W_c3,
                  b_c3.reshape(D_MODEL, D_MODEL))

    return (res, attn)


# speedup vs baseline: 14.0327x; 1.1743x over previous
import functools

import jax
import jax.numpy as jnp
from jax import lax
from jax.experimental import pallas as pl
from jax.experimental.pallas import tpu as pltpu
from jax.experimental.pallas import tpu_sc as plsc

B, N, D_POINTS, D_MODEL, K = 4, 1024, 32, 128, 36
BN = B * N
PAIRS = BN * K
R = 256
M = 128
PB = M * K
NC, NS = 2, 16
NW = NC * NS
PAIRS_B = N * K
PER_W = PAIRS_B // NW
CHUNK = 128
N_CHUNKS = PER_W // CHUNK

_f32 = jnp.float32
_dn = lambda c_lhs, c_rhs: (((c_lhs,), (c_rhs,)), ((), ()))


def _mm(a, b, cl=1, cr=1):
    return lax.dot_general(a, b, _dn(cl, cr), preferred_element_type=_f32)


def _proj_body(feat_ref, xyz_ref, wfc1_ref, bfc1_ref, wq_ref, wk_ref, wv_ref,
               wd1_ref, bd1_ref, x_ref, q_ref, kvq_ref, p_ref):
    feat = feat_ref[0]
    x = _mm(feat, wfc1_ref[...]) + bfc1_ref[...]
    q = _mm(x, wq_ref[...])
    k = _mm(x, wk_ref[...])
    v = _mm(x, wv_ref[...])
    x_ref[0] = x
    q_ref[0] = q
    xyz = xyz_ref[0]
    xyzmm = _mm(xyz, wd1_ref[...])
    kvq_ref[0] = jnp.concatenate([k, v, xyzmm], axis=1)
    p_ref[0] = xyzmm + bd1_ref[...]


def _run_proj(features, xyzp, wfc1, bfc1, wq, wk, wv, wd1p, bd1):
    spec_w = lambda shp: pl.BlockSpec(shp, lambda b: (0,) * len(shp))
    return pl.pallas_call(
        _proj_body,
        grid=(B,),
        in_specs=[
            pl.BlockSpec((1, N, D_POINTS), lambda b: (b, 0, 0)),
            pl.BlockSpec((1, N, 4), lambda b: (b, 0, 0)),
            spec_w((D_MODEL, D_POINTS)),
            spec_w((1, D_MODEL)),
            spec_w((D_MODEL, D_MODEL)),
            spec_w((D_MODEL, D_MODEL)),
            spec_w((D_MODEL, D_MODEL)),
            spec_w((D_MODEL, 4)),
            spec_w((1, D_MODEL)),
        ],
        out_specs=[
            pl.BlockSpec((1, N, D_MODEL), lambda b: (b, 0, 0)),
            pl.BlockSpec((1, N, D_MODEL), lambda b: (b, 0, 0)),
            pl.BlockSpec((1, N, 3 * D_MODEL), lambda b: (b, 0, 0)),
            pl.BlockSpec((1, N, D_MODEL), lambda b: (b, 0, 0)),
        ],
        out_shape=[
            jax.ShapeDtypeStruct((B, N, D_MODEL), _f32),
            jax.ShapeDtypeStruct((B, N, D_MODEL), _f32),
            jax.ShapeDtypeStruct((B, N, 3 * D_MODEL), _f32),
            jax.ShapeDtypeStruct((B, N, D_MODEL), _f32),
        ],
    )(features, xyzp, wfc1, bfc1, wq, wk, wv, wd1p, bd1)


def _topk_body(base, x_rows_ref, x_all_ref, out_ref):
    xr = x_rows_ref[...]
    xa = x_all_ref[...]
    t = lax.dot_general(xr.astype(jnp.bfloat16), xa.astype(jnp.bfloat16),
                        _dn(1, 1), preferred_element_type=_f32)
    sqr = jnp.sum(xr * xr, axis=1, keepdims=True)
    sqa = jnp.reshape(jnp.sum(xa * xa, axis=1, keepdims=True), (1, N))
    d = (-2.0 * t + sqr) + sqa
    iota = lax.broadcasted_iota(jnp.int32, (R, N), 1).astype(_f32)
    for t in range(K):
        m = jnp.min(d, axis=1, keepdims=True)
        sel = jnp.where(d == m, iota, float(N))
        idx = jnp.min(sel, axis=1, keepdims=True)
        out_ref[:, pl.ds(t, 1)] = idx.astype(jnp.int32) + base
        d = jnp.where(sel == idx, 3.4e38, d)


def _run_topk(xyzp_b, b):
    return pl.pallas_call(
        functools.partial(_topk_body, b * N),
        grid=(N // R,),
        in_specs=[
            pl.BlockSpec((R, 4), lambda rb: (rb, 0)),
            pl.BlockSpec((N, 4), lambda rb: (0, 0)),
        ],
        out_specs=pl.BlockSpec((R, K), lambda rb: (rb, 0)),
        out_shape=jax.ShapeDtypeStruct((N, K), jnp.int32),
    )(xyzp_b, xyzp_b)


def _sc_gather_body(kvq_hbm, idx_hbm, kvg_out, idx_v, rows_a, rows_b,
                    sem_a, sem_b, semw_a, semw_b):
    c = lax.axis_index("c")
    s = lax.axis_index("s")
    wid = s * NC + c
    base = wid * PER_W

    pltpu.sync_copy(idx_hbm.at[pl.ds(base, PER_W)], idx_v)
    bufs = [(rows_a, sem_a, semw_a), (rows_b, sem_b, semw_b)]

    def start(j):
        buf, sem, _ = bufs[j % 2]
        return pltpu.async_copy(
            kvq_hbm.at[idx_v.at[pl.ds(j * CHUNK, CHUNK)]], buf, sem)

    cps = {0: start(0), 1: start(1)}
    wrs = {}
    for j in range(N_CHUNKS):
        buf, _, semw = bufs[j % 2]
        cps[j].wait()
        wrs[j] = pltpu.async_copy(
            buf, kvg_out.at[pl.ds(base + j * CHUNK, CHUNK)], semw)
        if j + 2 < N_CHUNKS:
            wrs[j].wait()
            cps[j + 2] = start(j + 2)
    wrs[N_CHUNKS - 2].wait()
    wrs[N_CHUNKS - 1].wait()


def _run_sc_gather(kvq_tab, idx_flat):
    mesh = plsc.VectorSubcoreMesh(core_axis_name="c", subcore_axis_name="s")
    return pl.kernel(
        _sc_gather_body,
        out_type=jax.ShapeDtypeStruct((PAIRS_B, 3 * D_MODEL), _f32),
        mesh=mesh,
        scratch_types=[
            pltpu.VMEM((PER_W,), jnp.int32),
            pltpu.VMEM((CHUNK, 3 * D_MODEL), _f32),
            pltpu.VMEM((CHUNK, 3 * D_MODEL), _f32),
            pltpu.SemaphoreType.DMA,
            pltpu.SemaphoreType.DMA,
            pltpu.SemaphoreType.DMA,
            pltpu.SemaphoreType.DMA,
        ],
    )(kvq_tab, idx_flat)


def _pair_body(q_ref, p_ref, kvg_ref, wd2_ref,
               bd2_ref, wg1_ref, bg1_ref, wg2_ref, bg2_ref, attn_ref, res_ref):
    kvg = kvg_ref[...]
    kg = kvg[:, :D_MODEL]
    vg = kvg[:, D_MODEL:2 * D_MODEL]
    qg = kvg[:, 2 * D_MODEL:]
    p_rep = jnp.reshape(
        jnp.broadcast_to(p_ref[...][:, None, :], (M, K, D_MODEL)),
        (PB, D_MODEL))
    h1 = jnp.maximum(p_rep - qg, 0.0)
    pos = _mm(h1, wd2_ref[...]) + bd2_ref[...]
    q_rep = jnp.reshape(
        jnp.broadcast_to(q_ref[...][:, None, :], (M, K, D_MODEL)),
        (PB, D_MODEL))
    t = q_rep - kg + pos
    h2 = jnp.maximum(_mm(t, wg1_ref[...]) + bg1_ref[...], 0.0)
    attn = _mm(h2, wg2_ref[...]) + bg2_ref[...]
    den = jnp.sum(jnp.abs(attn) + 1e-5, axis=1, keepdims=True)
    attn = attn / den
    attn_ref[...] = jnp.reshape(attn, (M, K, D_MODEL))
    w = attn * (vg + pos)
    res_ref[...] = jnp.sum(jnp.reshape(w, (M, K, D_MODEL)), axis=1)


def _run_pair(q2, p2, kvg, wd2, bd2, wg1, bg1, wg2, bg2):
    spec_w = lambda shp: pl.BlockSpec(shp, lambda i: (0,) * len(shp))
    return pl.pallas_call(
        _pair_body,
        grid=(N // M,),
        in_specs=[
            pl.BlockSpec((M, D_MODEL), lambda i: (i, 0)),
            pl.BlockSpec((M, D_MODEL), lambda i: (i, 0)),
            pl.BlockSpec((PB, 3 * D_MODEL), lambda i: (i, 0)),
            spec_w((D_MODEL, D_MODEL)),
            spec_w((1, D_MODEL)),
            spec_w((D_MODEL, D_MODEL)),
            spec_w((1, D_MODEL)),
            spec_w((D_MODEL, D_MODEL)),
            spec_w((1, D_MODEL)),
        ],
        out_specs=[
            pl.BlockSpec((M, K, D_MODEL), lambda i: (i, 0, 0)),
            pl.BlockSpec((M, D_MODEL), lambda i: (i, 0)),
        ],
        out_shape=[
            jax.ShapeDtypeStruct((N, K, D_MODEL), _f32),
            jax.ShapeDtypeStruct((N, D_MODEL), _f32),
        ],
    )(q2, p2, kvg, wd2, bd2, wg1, bg1, wg2, bg2)


def _mt_body(resp_ref, x_ref, wc2_ref, bc2_ref, wc3_ref, bc3_ref, out_ref):
    feat0 = resp_ref[0]
    x = x_ref[0]
    ones = jnp.ones((1, N), _f32)

    def mtl(feat, wc, bc):
        m = _mm(ones, feat, 1, 0) * (1.0 / N)
        wv = _mm(m, wc)
        g = jnp.reshape(wv, (D_MODEL, D_MODEL)) + bc
        s = jnp.sum(jnp.abs(g) + 1e-5, axis=0, keepdims=True)
        return _mm(feat * (1.0 / s), g)

    r1 = mtl(feat0, wc2_ref[...], bc2_ref[...]) + x
    r2 = mtl(r1, wc3_ref[...], bc3_ref[...]) + r1
    out_ref[0] = r2


def _run_mt(res_pre, x, wc2, bc2r, wc3, bc3r):
    spec_w = lambda shp: pl.BlockSpec(shp, lambda b: (0,) * len(shp))
    return pl.pallas_call(
        _mt_body,
        grid=(B,),
        in_specs=[
            pl.BlockSpec((1, N, D_MODEL), lambda b: (b, 0, 0)),
            pl.BlockSpec((1, N, D_MODEL), lambda b: (b, 0, 0)),
            spec_w((D_MODEL * D_MODEL, D_MODEL)),
            spec_w((D_MODEL, D_MODEL)),
            spec_w((D_MODEL * D_MODEL, D_MODEL)),
            spec_w((D_MODEL, D_MODEL)),
        ],
        out_specs=pl.BlockSpec((1, N, D_MODEL), lambda b: (b, 0, 0)),
        out_shape=jax.ShapeDtypeStruct((B, N, D_MODEL), _f32),
    )(res_pre, x, wc2, bc2r, wc3, bc3r)


def kernel(features, xyz, W_fc1, b_fc1, W_c2, b_c2, W_c3, b_c3, W_d1, b_d1,
           W_d2, b_d2, W_g1, b_g1, W_g2, b_g2, W_q, W_k, W_v):
    xyzp = jnp.pad(xyz, ((0, 0), (0, 0), (0, 1)))
    wd1p = jnp.pad(W_d1, ((0, 0), (0, 1)))
    x, q, kvq, p = _run_proj(
        features, xyzp, W_fc1, b_fc1.reshape(1, -1), W_q, W_k, W_v,
        wd1p, b_d1.reshape(1, -1))

    kvq_tab = kvq.reshape(BN, 3 * D_MODEL)
    attn_parts, resp_parts = [], []
    for b in range(B):
        knn_b = _run_topk(xyzp[b], b)
        kvg_b = _run_sc_gather(kvq_tab, knn_b.reshape(PAIRS_B))
        attn_b, resp_b = _run_pair(
            q[b], p[b], kvg_b,
            W_d2, b_d2.reshape(1, -1),
            W_g1, b_g1.reshape(1, -1), W_g2, b_g2.reshape(1, -1))
        attn_parts.append(attn_b)
        resp_parts.append(resp_b)

    attn = jnp.stack(attn_parts, axis=0)
    res_pre = jnp.stack(resp_parts, axis=0)

    res = _run_mt(res_pre, x, W_c2,
                  b_c2.reshape(D_MODEL, D_MODEL), W_c3,
                  b_c3.reshape(D_MODEL, D_MODEL))

    return (res, attn)


# speedup vs baseline: 14.3074x; 1.0196x over previous
import functools

import jax
import jax.numpy as jnp
from jax import lax
from jax.experimental import pallas as pl
from jax.experimental.pallas import tpu as pltpu
from jax.experimental.pallas import tpu_sc as plsc

B, N, D_POINTS, D_MODEL, K = 4, 1024, 32, 128, 36
BN = B * N
PAIRS = BN * K
R = 256
M = 128
PB = M * K
NC, NS = 2, 16
NW = NC * NS
PAIRS_B = N * K
PER_W = PAIRS_B // NW
CHUNK = 128
N_CHUNKS = PER_W // CHUNK

_f32 = jnp.float32
_dn = lambda c_lhs, c_rhs: (((c_lhs,), (c_rhs,)), ((), ()))


def _mm(a, b, cl=1, cr=1):
    return lax.dot_general(a, b, _dn(cl, cr), preferred_element_type=_f32)


def _proj_body(feat_ref, xyz_ref, wfc1_ref, bfc1_ref, wq_ref, wk_ref, wv_ref,
               wd1_ref, bd1_ref, x_ref, q_ref, kvq_ref, p_ref):
    feat = feat_ref[0]
    x = _mm(feat, wfc1_ref[...]) + bfc1_ref[...]
    q = _mm(x, wq_ref[...])
    k = _mm(x, wk_ref[...])
    v = _mm(x, wv_ref[...])
    x_ref[0] = x
    q_ref[0] = q
    xyz = xyz_ref[0]
    xyzmm = _mm(xyz, wd1_ref[...])
    kvq_ref[0] = jnp.concatenate([k, v, xyzmm], axis=1)
    p_ref[0] = xyzmm + bd1_ref[...]


def _run_proj(features, xyzp, wfc1, bfc1, wq, wk, wv, wd1p, bd1):
    spec_w = lambda shp: pl.BlockSpec(shp, lambda b: (0,) * len(shp))
    return pl.pallas_call(
        _proj_body,
        grid=(B,),
        in_specs=[
            pl.BlockSpec((1, N, D_POINTS), lambda b: (b, 0, 0)),
            pl.BlockSpec((1, N, 4), lambda b: (b, 0, 0)),
            spec_w((D_MODEL, D_POINTS)),
            spec_w((1, D_MODEL)),
            spec_w((D_MODEL, D_MODEL)),
            spec_w((D_MODEL, D_MODEL)),
            spec_w((D_MODEL, D_MODEL)),
            spec_w((D_MODEL, 4)),
            spec_w((1, D_MODEL)),
        ],
        out_specs=[
            pl.BlockSpec((1, N, D_MODEL), lambda b: (b, 0, 0)),
            pl.BlockSpec((1, N, D_MODEL), lambda b: (b, 0, 0)),
            pl.BlockSpec((1, N, 3 * D_MODEL), lambda b: (b, 0, 0)),
            pl.BlockSpec((1, N, D_MODEL), lambda b: (b, 0, 0)),
        ],
        out_shape=[
            jax.ShapeDtypeStruct((B, N, D_MODEL), _f32),
            jax.ShapeDtypeStruct((B, N, D_MODEL), _f32),
            jax.ShapeDtypeStruct((B, N, 3 * D_MODEL), _f32),
            jax.ShapeDtypeStruct((B, N, D_MODEL), _f32),
        ],
    )(features, xyzp, wfc1, bfc1, wq, wk, wv, wd1p, bd1)


def _topk_body(base, x_rows_ref, x_all_ref, out_ref):
    xr = x_rows_ref[...]
    xa = x_all_ref[...]
    t = lax.dot_general(xr.astype(jnp.bfloat16), xa.astype(jnp.bfloat16),
                        _dn(1, 1), preferred_element_type=_f32)
    sqr = jnp.sum(xr * xr, axis=1, keepdims=True)
    sqa = jnp.reshape(jnp.sum(xa * xa, axis=1, keepdims=True), (1, N))
    d = (-2.0 * t + sqr) + sqa
    iota = lax.broadcasted_iota(jnp.int32, (R, N), 1).astype(_f32)
    for t in range(K):
        m = jnp.min(d, axis=1, keepdims=True)
        sel = jnp.where(d == m, iota, float(N))
        idx = jnp.min(sel, axis=1, keepdims=True)
        out_ref[:, pl.ds(t, 1)] = idx.astype(jnp.int32) + base
        d = jnp.where(sel == idx, 3.4e38, d)


def _run_topk(xyzp_b, b):
    return pl.pallas_call(
        functools.partial(_topk_body, b * N),
        grid=(N // R,),
        in_specs=[
            pl.BlockSpec((R, 4), lambda rb: (rb, 0)),
            pl.BlockSpec((N, 4), lambda rb: (0, 0)),
        ],
        out_specs=pl.BlockSpec((R, K), lambda rb: (rb, 0)),
        out_shape=jax.ShapeDtypeStruct((N, K), jnp.int32),
    )(xyzp_b, xyzp_b)


def _sc_gather_body(kvq_hbm, idx_hbm, kvg_out, idx_v, rows_a, rows_b,
                    sem_a, sem_b, semw_a, semw_b):
    c = lax.axis_index("c")
    s = lax.axis_index("s")
    wid = s * NC + c
    base = wid * PER_W

    pltpu.sync_copy(idx_hbm.at[pl.ds(base, PER_W)], idx_v)
    bufs = [(rows_a, sem_a, semw_a), (rows_b, sem_b, semw_b)]

    def start(j):
        buf, sem, _ = bufs[j % 2]
        return pltpu.async_copy(
            kvq_hbm.at[idx_v.at[pl.ds(j * CHUNK, CHUNK)]], buf, sem)

    cps = {0: start(0), 1: start(1)}
    wrs = {}
    for j in range(N_CHUNKS):
        buf, _, semw = bufs[j % 2]
        cps[j].wait()
        wrs[j] = pltpu.async_copy(
            buf, kvg_out.at[pl.ds(base + j * CHUNK, CHUNK)], semw)
        if j + 2 < N_CHUNKS:
            wrs[j].wait()
            cps[j + 2] = start(j + 2)
    wrs[N_CHUNKS - 2].wait()
    wrs[N_CHUNKS - 1].wait()


def _run_sc_gather(kvq_tab, idx_flat):
    mesh = plsc.VectorSubcoreMesh(core_axis_name="c", subcore_axis_name="s")
    return pl.kernel(
        _sc_gather_body,
        out_type=jax.ShapeDtypeStruct((PAIRS_B, 3 * D_MODEL), _f32),
        mesh=mesh,
        scratch_types=[
            pltpu.VMEM((PER_W,), jnp.int32),
            pltpu.VMEM((CHUNK, 3 * D_MODEL), _f32),
            pltpu.VMEM((CHUNK, 3 * D_MODEL), _f32),
            pltpu.SemaphoreType.DMA,
            pltpu.SemaphoreType.DMA,
            pltpu.SemaphoreType.DMA,
            pltpu.SemaphoreType.DMA,
        ],
    )(kvq_tab, idx_flat)


def _pair_body(q_ref, p_ref, kvg_ref, wd2_ref, bd2_ref, wg1_ref, bg1_ref,
               wg2_ref, bg2_ref, acc_ref, attn_ref, res_ref):
    del acc_ref
    kvg = kvg_ref[...]
    kg = kvg[:, :D_MODEL]
    vg = kvg[:, D_MODEL:2 * D_MODEL]
    qg = kvg[:, 2 * D_MODEL:]
    p_rep = jnp.reshape(
        jnp.broadcast_to(p_ref[...][:, None, :], (M, K, D_MODEL)),
        (PB, D_MODEL))
    h1 = jnp.maximum(p_rep - qg, 0.0)
    pos = _mm(h1, wd2_ref[...]) + bd2_ref[...]
    q_rep = jnp.reshape(
        jnp.broadcast_to(q_ref[...][:, None, :], (M, K, D_MODEL)),
        (PB, D_MODEL))
    t = q_rep - kg + pos
    h2 = jnp.maximum(_mm(t, wg1_ref[...]) + bg1_ref[...], 0.0)
    attn = _mm(h2, wg2_ref[...]) + bg2_ref[...]
    den = jnp.sum(jnp.abs(attn) + 1e-5, axis=1, keepdims=True)
    attn = attn / den
    attn_ref[0] = jnp.reshape(attn, (M, K, D_MODEL))
    w = attn * (vg + pos)
    res_ref[...] = jnp.sum(jnp.reshape(w, (M, K, D_MODEL)), axis=1)


def _run_pair(q2, p2, kvg, wd2, bd2, wg1, bg1, wg2, bg2, attn_acc, b):
    spec_w = lambda shp: pl.BlockSpec(shp, lambda i: (0,) * len(shp))
    return pl.pallas_call(
        _pair_body,
        grid=(N // M,),
        in_specs=[
            pl.BlockSpec((M, D_MODEL), lambda i: (i, 0)),
            pl.BlockSpec((M, D_MODEL), lambda i: (i, 0)),
            pl.BlockSpec((PB, 3 * D_MODEL), lambda i: (i, 0)),
            spec_w((D_MODEL, D_MODEL)),
            spec_w((1, D_MODEL)),
            spec_w((D_MODEL, D_MODEL)),
            spec_w((1, D_MODEL)),
            spec_w((D_MODEL, D_MODEL)),
            spec_w((1, D_MODEL)),
            pl.BlockSpec((1, 8, K, D_MODEL), lambda i: (0, 0, 0, 0)),
        ],
        out_specs=[
            pl.BlockSpec((1, M, K, D_MODEL), lambda i: (b, i, 0, 0)),
            pl.BlockSpec((M, D_MODEL), lambda i: (i, 0)),
        ],
        out_shape=[
            jax.ShapeDtypeStruct((B, N, K, D_MODEL), _f32),
            jax.ShapeDtypeStruct((N, D_MODEL), _f32),
        ],
        input_output_aliases={9: 0},
    )(q2, p2, kvg, wd2, bd2, wg1, bg1, wg2, bg2, attn_acc)


def _mt_body(resp_ref, x_ref, wc2_ref, bc2_ref, wc3_ref, bc3_ref, out_ref):
    feat0 = resp_ref[0]
    x = x_ref[0]
    ones = jnp.ones((1, N), _f32)

    def mtl(feat, wc, bc):
        m = _mm(ones, feat, 1, 0) * (1.0 / N)
        wv = _mm(m, wc)
        g = jnp.reshape(wv, (D_MODEL, D_MODEL)) + bc
        s = jnp.sum(jnp.abs(g) + 1e-5, axis=0, keepdims=True)
        return _mm(feat * (1.0 / s), g)

    r1 = mtl(feat0, wc2_ref[...], bc2_ref[...]) + x
    r2 = mtl(r1, wc3_ref[...], bc3_ref[...]) + r1
    out_ref[0] = r2


def _run_mt(res_pre, x, wc2, bc2r, wc3, bc3r):
    spec_w = lambda shp: pl.BlockSpec(shp, lambda b: (0,) * len(shp))
    return pl.pallas_call(
        _mt_body,
        grid=(B,),
        in_specs=[
            pl.BlockSpec((1, N, D_MODEL), lambda b: (b, 0, 0)),
            pl.BlockSpec((1, N, D_MODEL), lambda b: (b, 0, 0)),
            spec_w((D_MODEL * D_MODEL, D_MODEL)),
            spec_w((D_MODEL, D_MODEL)),
            spec_w((D_MODEL * D_MODEL, D_MODEL)),
            spec_w((D_MODEL, D_MODEL)),
        ],
        out_specs=pl.BlockSpec((1, N, D_MODEL), lambda b: (b, 0, 0)),
        out_shape=jax.ShapeDtypeStruct((B, N, D_MODEL), _f32),
    )(res_pre, x, wc2, bc2r, wc3, bc3r)


def kernel(features, xyz, W_fc1, b_fc1, W_c2, b_c2, W_c3, b_c3, W_d1, b_d1,
           W_d2, b_d2, W_g1, b_g1, W_g2, b_g2, W_q, W_k, W_v):
    xyzp = jnp.pad(xyz, ((0, 0), (0, 0), (0, 1)))
    wd1p = jnp.pad(W_d1, ((0, 0), (0, 1)))
    x, q, kvq, p = _run_proj(
        features, xyzp, W_fc1, b_fc1.reshape(1, -1), W_q, W_k, W_v,
        wd1p, b_d1.reshape(1, -1))

    kvq_tab = kvq.reshape(BN, 3 * D_MODEL)
    knns = [_run_topk(xyzp[b], b) for b in range(B)]
    kvgs = [_run_sc_gather(kvq_tab, knns[b].reshape(PAIRS_B))
            for b in range(B)]
    attn = jnp.zeros((B, N, K, D_MODEL), _f32)
    resp_parts = []
    for b in range(B):
        attn, resp_b = _run_pair(
            q[b], p[b], kvgs[b],
            W_d2, b_d2.reshape(1, -1),
            W_g1, b_g1.reshape(1, -1), W_g2, b_g2.reshape(1, -1),
            attn, b)
        resp_parts.append(resp_b)

    res_pre = jnp.stack(resp_parts, axis=0)

    res = _run_mt(res_pre, x, W_c2,
                  b_c2.reshape(D_MODEL, D_MODEL), W_c3,
                  b_c3.reshape(D_MODEL, D_MODEL))

    return (res, attn)


# speedup vs baseline: 15.2743x; 1.0676x over previous
import functools

import jax
import jax.numpy as jnp
from jax import lax
from jax.experimental import pallas as pl
from jax.experimental.pallas import tpu as pltpu
from jax.experimental.pallas import tpu_sc as plsc

B, N, D_POINTS, D_MODEL, K = 4, 1024, 32, 128, 36
BN = B * N
PAIRS = BN * K
R = 256
M = 128
PB = M * K
NC, NS = 2, 16
NW = NC * NS
PAIRS_B = N * K
PER_W = PAIRS_B // NW
CHUNK = 128
N_CHUNKS = PER_W // CHUNK

_f32 = jnp.float32
_dn = lambda c_lhs, c_rhs: (((c_lhs,), (c_rhs,)), ((), ()))


def _mm(a, b, cl=1, cr=1):
    return lax.dot_general(a, b, _dn(cl, cr), preferred_element_type=_f32)


def _proj_body(feat_ref, xyz_ref, wfc1_ref, bfc1_ref, wq_ref, wk_ref, wv_ref,
               wd1_ref, bd1_ref, x_ref, q_ref, kvq_ref, p_ref):
    feat = feat_ref[0]
    x = _mm(feat, wfc1_ref[...]) + bfc1_ref[...]
    q = _mm(x, wq_ref[...])
    k = _mm(x, wk_ref[...])
    v = _mm(x, wv_ref[...])
    x_ref[0] = x
    q_ref[0] = q
    xyz = xyz_ref[0]
    xyzmm = _mm(xyz, wd1_ref[...])
    kvq_ref[0] = jnp.concatenate([k, v, xyzmm], axis=1)
    p_ref[0] = xyzmm + bd1_ref[...]


def _run_proj(features, xyzp, wfc1, bfc1, wq, wk, wv, wd1p, bd1):
    spec_w = lambda shp: pl.BlockSpec(shp, lambda b: (0,) * len(shp))
    return pl.pallas_call(
        _proj_body,
        grid=(B,),
        in_specs=[
            pl.BlockSpec((1, N, D_POINTS), lambda b: (b, 0, 0)),
            pl.BlockSpec((1, N, 4), lambda b: (b, 0, 0)),
            spec_w((D_MODEL, D_POINTS)),
            spec_w((1, D_MODEL)),
            spec_w((D_MODEL, D_MODEL)),
            spec_w((D_MODEL, D_MODEL)),
            spec_w((D_MODEL, D_MODEL)),
            spec_w((D_MODEL, 4)),
            spec_w((1, D_MODEL)),
        ],
        out_specs=[
            pl.BlockSpec((1, N, D_MODEL), lambda b: (b, 0, 0)),
            pl.BlockSpec((1, N, D_MODEL), lambda b: (b, 0, 0)),
            pl.BlockSpec((1, N, 3 * D_MODEL), lambda b: (b, 0, 0)),
            pl.BlockSpec((1, N, D_MODEL), lambda b: (b, 0, 0)),
        ],
        out_shape=[
            jax.ShapeDtypeStruct((B, N, D_MODEL), _f32),
            jax.ShapeDtypeStruct((B, N, D_MODEL), _f32),
            jax.ShapeDtypeStruct((B, N, 3 * D_MODEL), _f32),
            jax.ShapeDtypeStruct((B, N, D_MODEL), _f32),
        ],
    )(features, xyzp, wfc1, bfc1, wq, wk, wv, wd1p, bd1)


def _topk_body(base, x_rows_ref, x_all_ref, out_ref):
    xr = x_rows_ref[...]
    xa = x_all_ref[...]
    t = lax.dot_general(xr.astype(jnp.bfloat16), xa.astype(jnp.bfloat16),
                        _dn(1, 1), preferred_element_type=_f32)
    sqr = jnp.sum(xr * xr, axis=1, keepdims=True)
    sqa = jnp.reshape(jnp.sum(xa * xa, axis=1, keepdims=True), (1, N))
    d = (-2.0 * t + sqr) + sqa
    iota = lax.broadcasted_iota(jnp.int32, (R, N), 1).astype(_f32)
    for t in range(K):
        m = jnp.min(d, axis=1, keepdims=True)
        sel = jnp.where(d == m, iota, float(N))
        idx = jnp.min(sel, axis=1, keepdims=True)
        out_ref[:, pl.ds(t, 1)] = idx.astype(jnp.int32) + base
        d = jnp.where(sel == idx, 3.4e38, d)


def _run_topk(xyzp_b, b):
    return pl.pallas_call(
        functools.partial(_topk_body, b * N),
        grid=(N // R,),
        in_specs=[
            pl.BlockSpec((R, 4), lambda rb: (rb, 0)),
            pl.BlockSpec((N, 4), lambda rb: (0, 0)),
        ],
        out_specs=pl.BlockSpec((R, K), lambda rb: (rb, 0)),
        out_shape=jax.ShapeDtypeStruct((N, K), jnp.int32),
    )(xyzp_b, xyzp_b)


def _sc_gather_body(kvq_hbm, idx_hbm, kvg_out, idx_v, rows_a, rows_b,
                    sem_a, sem_b, semw_a, semw_b):
    c = lax.axis_index("c")
    s = lax.axis_index("s")
    wid = s * NC + c
    base = wid * PER_W

    pltpu.sync_copy(idx_hbm.at[pl.ds(base, PER_W)], idx_v)
    bufs = [(rows_a, sem_a, semw_a), (rows_b, sem_b, semw_b)]

    def start(j):
        buf, sem, _ = bufs[j % 2]
        return pltpu.async_copy(
            kvq_hbm.at[idx_v.at[pl.ds(j * CHUNK, CHUNK)]], buf, sem)

    cps = {0: start(0), 1: start(1)}
    wrs = {}
    for j in range(N_CHUNKS):
        buf, _, semw = bufs[j % 2]
        cps[j].wait()
        wrs[j] = pltpu.async_copy(
            buf, kvg_out.at[pl.ds(base + j * CHUNK, CHUNK)], semw)
        if j + 2 < N_CHUNKS:
            wrs[j].wait()
            cps[j + 2] = start(j + 2)
    wrs[N_CHUNKS - 2].wait()
    wrs[N_CHUNKS - 1].wait()


def _run_sc_gather(kvq_tab, idx_flat):
    mesh = plsc.VectorSubcoreMesh(core_axis_name="c", subcore_axis_name="s")
    return pl.kernel(
        _sc_gather_body,
        out_type=jax.ShapeDtypeStruct((PAIRS_B, 3 * D_MODEL), _f32),
        mesh=mesh,
        scratch_types=[
            pltpu.VMEM((PER_W,), jnp.int32),
            pltpu.VMEM((CHUNK, 3 * D_MODEL), _f32),
            pltpu.VMEM((CHUNK, 3 * D_MODEL), _f32),
            pltpu.SemaphoreType.DMA,
            pltpu.SemaphoreType.DMA,
            pltpu.SemaphoreType.DMA,
            pltpu.SemaphoreType.DMA,
        ],
    )(kvq_tab, idx_flat)


def _pair_body(q_ref, p_ref, kvg_ref, wd2_ref, bd2_ref, wg1_ref, bg1_ref,
               wg2_ref, bg2_ref, *refs):
    if len(refs) == 3:
        acc_ref, attn_ref, res_ref = refs
        del acc_ref
    else:
        attn_ref, res_ref = refs
    kvg = kvg_ref[...]
    kg = kvg[:, :D_MODEL]
    vg = kvg[:, D_MODEL:2 * D_MODEL]
    qg = kvg[:, 2 * D_MODEL:]
    p_rep = jnp.reshape(
        jnp.broadcast_to(p_ref[...][:, None, :], (M, K, D_MODEL)),
        (PB, D_MODEL))
    h1 = jnp.maximum(p_rep - qg, 0.0)
    pos = _mm(h1, wd2_ref[...]) + bd2_ref[...]
    q_rep = jnp.reshape(
        jnp.broadcast_to(q_ref[...][:, None, :], (M, K, D_MODEL)),
        (PB, D_MODEL))
    t = q_rep - kg + pos
    h2 = jnp.maximum(_mm(t, wg1_ref[...]) + bg1_ref[...], 0.0)
    attn = _mm(h2, wg2_ref[...]) + bg2_ref[...]
    den = jnp.sum(jnp.abs(attn) + 1e-5, axis=1, keepdims=True)
    attn = attn / den
    attn_ref[0] = jnp.reshape(attn, (M, K, D_MODEL))
    w = attn * (vg + pos)
    res_ref[...] = jnp.sum(jnp.reshape(w, (M, K, D_MODEL)), axis=1)


def _run_pair(q2, p2, kvg, wd2, bd2, wg1, bg1, wg2, bg2, attn_acc, b):
    spec_w = lambda shp: pl.BlockSpec(shp, lambda i: (0,) * len(shp))
    in_specs = [
        pl.BlockSpec((M, D_MODEL), lambda i: (i, 0)),
        pl.BlockSpec((M, D_MODEL), lambda i: (i, 0)),
        pl.BlockSpec((PB, 3 * D_MODEL), lambda i: (i, 0)),
        spec_w((D_MODEL, D_MODEL)),
        spec_w((1, D_MODEL)),
        spec_w((D_MODEL, D_MODEL)),
        spec_w((1, D_MODEL)),
        spec_w((D_MODEL, D_MODEL)),
        spec_w((1, D_MODEL)),
    ]
    args = [q2, p2, kvg, wd2, bd2, wg1, bg1, wg2, bg2]
    aliases = {}
    if attn_acc is not None:
        in_specs.append(pl.BlockSpec((1, 8, K, D_MODEL),
                                     lambda i: (0, 0, 0, 0)))
        args.append(attn_acc)
        aliases = {9: 0}
    return pl.pallas_call(
        _pair_body,
        grid=(N // M,),
        in_specs=in_specs,
        out_specs=[
            pl.BlockSpec((1, M, K, D_MODEL), lambda i: (b, i, 0, 0)),
            pl.BlockSpec((M, D_MODEL), lambda i: (i, 0)),
        ],
        out_shape=[
            jax.ShapeDtypeStruct((B, N, K, D_MODEL), _f32),
            jax.ShapeDtypeStruct((N, D_MODEL), _f32),
        ],
        input_output_aliases=aliases,
    )(*args)


def _mt_body(resp_ref, x_ref, wc2_ref, bc2_ref, wc3_ref, bc3_ref, out_ref):
    feat0 = resp_ref[0]
    x = x_ref[0]
    ones = jnp.ones((1, N), _f32)

    def mtl(feat, wc, bc):
        m = _mm(ones, feat, 1, 0) * (1.0 / N)
        wv = _mm(m, wc)
        g = jnp.reshape(wv, (D_MODEL, D_MODEL)) + bc
        s = jnp.sum(jnp.abs(g) + 1e-5, axis=0, keepdims=True)
        return _mm(feat * (1.0 / s), g)

    r1 = mtl(feat0, wc2_ref[...], bc2_ref[...]) + x
    r2 = mtl(r1, wc3_ref[...], bc3_ref[...]) + r1
    out_ref[0] = r2


def _run_mt(res_pre, x, wc2, bc2r, wc3, bc3r):
    spec_w = lambda shp: pl.BlockSpec(shp, lambda b: (0,) * len(shp))
    return pl.pallas_call(
        _mt_body,
        grid=(B,),
        in_specs=[
            pl.BlockSpec((1, N, D_MODEL), lambda b: (b, 0, 0)),
            pl.BlockSpec((1, N, D_MODEL), lambda b: (b, 0, 0)),
            spec_w((D_MODEL * D_MODEL, D_MODEL)),
            spec_w((D_MODEL, D_MODEL)),
            spec_w((D_MODEL * D_MODEL, D_MODEL)),
            spec_w((D_MODEL, D_MODEL)),
        ],
        out_specs=pl.BlockSpec((1, N, D_MODEL), lambda b: (b, 0, 0)),
        out_shape=jax.ShapeDtypeStruct((B, N, D_MODEL), _f32),
    )(res_pre, x, wc2, bc2r, wc3, bc3r)


def kernel(features, xyz, W_fc1, b_fc1, W_c2, b_c2, W_c3, b_c3, W_d1, b_d1,
           W_d2, b_d2, W_g1, b_g1, W_g2, b_g2, W_q, W_k, W_v):
    xyzp = jnp.pad(xyz, ((0, 0), (0, 0), (0, 1)))
    wd1p = jnp.pad(W_d1, ((0, 0), (0, 1)))
    x, q, kvq, p = _run_proj(
        features, xyzp, W_fc1, b_fc1.reshape(1, -1), W_q, W_k, W_v,
        wd1p, b_d1.reshape(1, -1))

    kvq_tab = kvq.reshape(BN, 3 * D_MODEL)
    knns = [_run_topk(xyzp[b], b) for b in range(B)]
    kvgs = [_run_sc_gather(kvq_tab, knns[b].reshape(PAIRS_B))
            for b in range(B)]
    attn = None
    resp_parts = []
    for b in range(B):
        attn, resp_b = _run_pair(
            q[b], p[b], kvgs[b],
            W_d2, b_d2.reshape(1, -1),
            W_g1, b_g1.reshape(1, -1), W_g2, b_g2.reshape(1, -1),
            attn, b)
        resp_parts.append(resp_b)

    res_pre = jnp.stack(resp_parts, axis=0)

    res = _run_mt(res_pre, x, W_c2,
                  b_c2.reshape(D_MODEL, D_MODEL), W_c3,
                  b_c3.reshape(D_MODEL, D_MODEL))

    return (res, attn)
